# baseline (speedup 1.0000x reference)
import jax
import jax.numpy as jnp
from jax import lax
from jax.experimental import pallas as pl
from jax.experimental.pallas import tpu as pltpu
from jax.experimental.pallas import tpu_sc as plsc

NC = 2
NS = 16
NW = NC * NS
LANES = 16
CHUNK = 4000
CCHUNK = 2000


def _sc_mesh():
    return plsc.VectorSubcoreMesh(
        core_axis_name="c", subcore_axis_name="s", num_cores=NC, num_subcores=NS
    )


def _full(s):
    return jnp.full((LANES,), s, jnp.float32)


def _make_sc_conv(n_nodes, n_edges, cpt, with_count):
    tw = n_nodes * cpt
    nch = n_edges // CHUNK
    assert nch % 2 == 0

    out_type = [jax.ShapeDtypeStruct((NW * tw,), jnp.float32)]
    scratch = [
        pltpu.VMEM((tw,), jnp.float32),
        pltpu.VMEM((tw,), jnp.float32),
        pltpu.VMEM((16,), jnp.float32),
    ]
    for _ in range(2):
        scratch += [pltpu.VMEM((CHUNK,), jnp.int32),
                    pltpu.VMEM((CHUNK,), jnp.int32),
                    pltpu.VMEM((2 * CHUNK,), jnp.float32)]
    scratch += [pltpu.SemaphoreType.DMA, pltpu.SemaphoreType.DMA]
    if with_count:
        out_type.append(jax.ShapeDtypeStruct((NW * n_nodes,), jnp.float32))
        scratch.append(pltpu.VMEM((n_nodes,), jnp.float32))

    def body(pt_hbm, src_hbm, dst_hbm, ew_hbm, wcb_hbm, *rest):
        if with_count:
            (acc_hbm, cnt_hbm, table_v, acc_v, wcb_v,
             sA, dA, eA, sB, dB, eB, semA, semB, cnt_v) = rest
        else:
            (acc_hbm, table_v, acc_v, wcb_v,
             sA, dA, eA, sB, dB, eB, semA, semB) = rest
        bufA = (sA, dA, eA)
        bufB = (sB, dB, eB)
        wid = lax.axis_index("s") * NC + lax.axis_index("c")
        pltpu.sync_copy(pt_hbm.at[pl.ds(wid * tw, tw)], table_v)
        pltpu.sync_copy(wcb_hbm.at[pl.ds(wid * 16, 16)], wcb_v)
        zero = jnp.zeros((LANES,), jnp.float32)

        @plsc.parallel_loop(0, tw, step=LANES, unroll=8)
        def _(i):
            acc_v[pl.ds(i, LANES)] = zero

        wrow = wcb_v[...]
        w0 = [_full(wrow[c]) for c in range(cpt)]
        w1 = [_full(wrow[8 + c]) for c in range(cpt)]
        lane2 = lax.iota(jnp.int32, LANES) * 2

        def fire(off, bufs, sem):
            pltpu.async_copy(src_hbm.at[pl.ds(off, CHUNK)], bufs[0], sem)
            pltpu.async_copy(dst_hbm.at[pl.ds(off, CHUNK)], bufs[1], sem)
            pltpu.async_copy(ew_hbm.at[pl.ds(2 * off, 2 * CHUNK)], bufs[2], sem)

        def drain(bufs, sem):
            pltpu.make_async_copy(src_hbm.at[pl.ds(0, CHUNK)], bufs[0], sem).wait()
            pltpu.make_async_copy(dst_hbm.at[pl.ds(0, CHUNK)], bufs[1], sem).wait()
            pltpu.make_async_copy(ew_hbm.at[pl.ds(0, 2 * CHUNK)], bufs[2], sem).wait()

        def process(bufs):
            @plsc.parallel_loop(0, CHUNK, step=LANES, unroll=8)
            def _(b):
                sv = bufs[0][pl.ds(b, LANES)]
                dv = bufs[1][pl.ds(b, LANES)]
                ei = lane2 + 2 * b
                e0 = plsc.load_gather(bufs[2], [ei])
                e1 = plsc.load_gather(bufs[2], [ei + 1])
                for c in range(cpt):
                    gth = plsc.load_gather(table_v, [sv + (c * n_nodes)])
                    m = jnp.maximum(gth + (e0 * w0[c] + e1 * w1[c]), 0.0)
                    plsc.addupdate_scatter(acc_v, [dv + (c * n_nodes)], m)

        last = (nch - 1) * CHUNK
        fire(0, bufA, semA)

        @pl.loop(0, nch // 2)
        def _(gp):
            g0 = gp * 2
            fire(jnp.minimum((g0 + 1) * CHUNK, last), bufB, semB)
            drain(bufA, semA)
            process(bufA)
            fire(jnp.minimum((g0 + 2) * CHUNK, last), bufA, semA)
            drain(bufB, semB)
            process(bufB)

        drain(bufA, semA)

        if with_count:
            @plsc.parallel_loop(0, n_nodes, step=LANES, unroll=8)
            def _(i):
                cnt_v[pl.ds(i, LANES)] = zero

            epw = n_edges // NW
            base = wid * epw
            ones = jnp.ones((LANES,), jnp.float32)

            @pl.loop(0, epw // CCHUNK)
            def _(g):
                db = dA.at[pl.ds(0, CCHUNK)]
                pltpu.sync_copy(dst_hbm.at[pl.ds(base + g * CCHUNK, CCHUNK)], db)

                @plsc.parallel_loop(0, CCHUNK, step=LANES, unroll=4)
                def _(i):
                    plsc.addupdate_scatter(cnt_v, [dA[pl.ds(i, LANES)]], ones)

            pltpu.sync_copy(cnt_v, cnt_hbm.at[pl.ds(wid * n_nodes, n_nodes)])

        pltpu.sync_copy(acc_v, acc_hbm.at[pl.ds(wid * tw, tw)])

    return pl.kernel(
        body,
        out_type=tuple(out_type) if with_count else out_type[0],
        mesh=_sc_mesh(),
        scratch_types=tuple(scratch),
        compiler_params=pltpu.CompilerParams(needs_layout_passes=False),
    )


def _make_sc_pet(n_nodes, n_edges):
    cpt = 4
    tw = n_nodes * cpt
    nch = n_edges // CHUNK
    assert nch % 2 == 0

    def body(xt_hbm, src_hbm, dst_hbm, wet_hbm, out_hbm,
             table_v, wet_v, sA, dA, sB, dB, oA, oB, semA, semB, semWA, semWB):
        bufA = (sA, dA)
        bufB = (sB, dB)
        srcs = (src_hbm, dst_hbm)
        wid = lax.axis_index("s") * NC + lax.axis_index("c")
        obase = wid * n_edges
        pltpu.sync_copy(xt_hbm.at[pl.ds(wid * tw, tw)], table_v)
        pltpu.sync_copy(wet_hbm.at[pl.ds(wid * 16, 16)], wet_v)
        wetrow = wet_v[...]
        wv = [_full(wetrow[c]) for c in range(cpt)]

        def fire(off, bufs, sem):
            for hb, b in zip(srcs, bufs):
                pltpu.async_copy(hb.at[pl.ds(off, CHUNK)], b, sem)

        def drain(bufs, sem):
            for hb, b in zip(srcs, bufs):
                pltpu.make_async_copy(hb.at[pl.ds(0, CHUNK)], b, sem).wait()

        def process(bufs, ob):
            @plsc.parallel_loop(0, CHUNK, step=LANES, unroll=4)
            def _(b):
                sv = bufs[0][pl.ds(b, LANES)]
                dv = bufs[1][pl.ds(b, LANES)]
                acc = jnp.zeros((LANES,), jnp.float32)
                for c in range(cpt):
                    a = plsc.load_gather(table_v, [sv + (c * n_nodes)])
                    bb = plsc.load_gather(table_v, [dv + (c * n_nodes)])
                    acc = acc + jnp.abs(a - bb) * wv[c]
                ob[pl.ds(b, LANES)] = acc

        def wdrain(ob, semw):
            pltpu.make_async_copy(src_hbm.at[pl.ds(0, CHUNK)], ob, semw).wait()

        last = (nch - 1) * CHUNK
        fire(0, bufA, semA)
        pltpu.async_copy(oA, out_hbm.at[pl.ds(obase, CHUNK)], semWA)
        pltpu.async_copy(oB, out_hbm.at[pl.ds(obase + CHUNK, CHUNK)], semWB)

        @pl.loop(0, nch // 2)
        def _(gp):
            g0 = gp * 2
            fire(jnp.minimum((g0 + 1) * CHUNK, last), bufB, semB)
            drain(bufA, semA)
            wdrain(oA, semWA)
            process(bufA, oA)
            pltpu.async_copy(oA, out_hbm.at[pl.ds(obase + g0 * CHUNK, CHUNK)], semWA)
            fire(jnp.minimum((g0 + 2) * CHUNK, last), bufA, semA)
            drain(bufB, semB)
            wdrain(oB, semWB)
            process(bufB, oB)
            pltpu.async_copy(oB, out_hbm.at[pl.ds(obase + (g0 + 1) * CHUNK, CHUNK)], semWB)

        drain(bufA, semA)
        wdrain(oA, semWA)
        wdrain(oB, semWB)

    return pl.kernel(
        body,
        out_type=jax.ShapeDtypeStruct((NW * n_edges,), jnp.float32),
        mesh=_sc_mesh(),
        scratch_types=(
            pltpu.VMEM((tw,), jnp.float32),
            pltpu.VMEM((16,), jnp.float32),
            pltpu.VMEM((CHUNK,), jnp.int32),
            pltpu.VMEM((CHUNK,), jnp.int32),
            pltpu.VMEM((CHUNK,), jnp.int32),
            pltpu.VMEM((CHUNK,), jnp.int32),
            pltpu.VMEM((CHUNK,), jnp.float32),
            pltpu.VMEM((CHUNK,), jnp.float32),
            pltpu.SemaphoreType.DMA, pltpu.SemaphoreType.DMA,
            pltpu.SemaphoreType.DMA, pltpu.SemaphoreType.DMA,
        ),
        compiler_params=pltpu.CompilerParams(needs_layout_passes=False),
    )


_TC_PARAMS = pltpu.CompilerParams(vmem_limit_bytes=100 * 1024 * 1024)


def _mm_tt(a, b):
    return lax.dot_general(a, b, (((0,), (0,)), ((), ())),
                           preferred_element_type=jnp.float32)


def _inv_cnt(cnt_ref):
    c = jnp.sum(cnt_ref[...], axis=0, keepdims=True)
    return 1.0 / jnp.maximum(c, 1.0)


def _tc_call(body, out_shapes, *args):
    outs = [jax.ShapeDtypeStruct(s, jnp.float32) for s in out_shapes]
    return pl.pallas_call(
        body,
        out_shape=outs[0] if len(outs) == 1 else outs,
        compiler_params=_TC_PARAMS,
    )(*args)


def _tc_project_t(x, wc, bc):
    n, f = x.shape
    dout = wc.shape[1]

    def body(x_ref, wc_ref, bc_ref, o_ref, xt_ref):
        xt = x_ref[...].T
        xt_ref[...] = xt
        o_ref[...] = _mm_tt(wc_ref[...], xt) + bc_ref[...]

    return _tc_call(body, [(dout, n), (f, n)], x, wc, bc.reshape(-1, 1))


def _tc_layer_t(acct, cnt, w, b, wc, bc):
    n = acct.shape[1]
    dout = wc.shape[1]

    def body(acc_ref, cnt_ref, w_ref, b_ref, wc_ref, bc_ref, o_ref):
        aggt = acc_ref[...] * _inv_cnt(cnt_ref)
        ht = jnp.maximum(_mm_tt(w_ref[...], aggt) + b_ref[...], 0.0)
        o_ref[...] = _mm_tt(wc_ref[...], ht) + bc_ref[...]

    return _tc_call(body, [(dout, n)], acct, cnt, w, b.reshape(-1, 1),
                    wc, bc.reshape(-1, 1))


def _tc_layer2_t(acct, cnt, w, b, wc, bc, latent):
    n = acct.shape[1]
    dmid = w.shape[1]
    dout = wc.shape[1]

    def body(acc_ref, cnt_ref, w_ref, b_ref, wc_ref, bc_ref, h_ref, p_ref):
        aggt = acc_ref[...] * _inv_cnt(cnt_ref)
        ht = jnp.maximum(_mm_tt(w_ref[...], aggt) + b_ref[...], 0.0)
        h_ref[...] = ht.T
        p_ref[...] = _mm_tt(wc_ref[...], ht[:latent]) + bc_ref[...]

    return _tc_call(body, [(n, dmid), (dout, n)], acct, cnt, w,
                    b.reshape(-1, 1), wc, bc.reshape(-1, 1))


def _tc_final(acct, cnt, w, b):
    n = acct.shape[1]
    dout = w.shape[1]

    def body(acc_ref, cnt_ref, w_ref, b_ref, o_ref):
        aggt = acc_ref[...] * _inv_cnt(cnt_ref)
        o_ref[...] = jnp.tanh(_mm_tt(aggt, w_ref[...]) + b_ref[...])

    return _tc_call(body, [(n, dout)], acct, cnt, w, b.reshape(1, -1))


def _tc_pet(parts, ewr, sv):
    e = parts.shape[1]
    be = 12800

    def body(p_ref, ew_ref, s_ref, o_ref):
        s = jnp.sum(p_ref[...], axis=0, keepdims=True)
        o_ref[...] = (s + ew_ref[0:1, :] * s_ref[0, 0]
                      + ew_ref[1:2, :] * s_ref[0, 1] + s_ref[0, 2])

    return pl.pallas_call(
        body,
        grid=(e // be,),
        in_specs=[pl.BlockSpec((NW, be), lambda i: (0, i)),
                  pl.BlockSpec((2, be), lambda i: (0, i)),
                  pl.BlockSpec((1, 128), lambda i: (0, 0))],
        out_specs=pl.BlockSpec((1, be), lambda i: (0, i)),
        out_shape=jax.ShapeDtypeStruct((1, e), jnp.float32),
        compiler_params=_TC_PARAMS,
    )(parts, ewr, sv)


def _pack_wcb(wc, din, cpt):
    bot = wc[din:]
    b0 = bot[0].reshape(NW, cpt)
    b1 = bot[1].reshape(NW, cpt)
    out = jnp.zeros((NW, 16), jnp.float32)
    return out.at[:, :cpt].set(b0).at[:, 8:8 + cpt].set(b1).reshape(-1)


def kernel(x, edge_index, edge_weight, W1, b1, Wc1, bc1, W2, b2, Wc2, bc2,
           W3, b3, Wc3, bc3, W4, b4, Wc4, bc4, Wet, bet):
    n, f = x.shape
    e = edge_index.shape[1]
    h = W1.shape[1]
    l = W3.shape[0]

    src = edge_index[0]
    dst = edge_index[1]
    ew_flat = edge_weight.reshape(-1)

    sc_conv_first = _make_sc_conv(n, e, f // NW, with_count=True)
    sc_conv = _make_sc_conv(n, e, f // NW, with_count=False)
    sc_conv_l = _make_sc_conv(n, e, l // NW, with_count=False)
    sc_pet = _make_sc_pet(n, e)

    p1t, xt = _tc_project_t(x, Wc1[:f], bc1)
    acc1, cntp = sc_conv_first(p1t.reshape(-1), src, dst, ew_flat,
                               _pack_wcb(Wc1, f, f // NW))
    cnt = cntp.reshape(NW, n)

    wet_pack = jnp.zeros((NW, 16), jnp.float32).at[:, :4].set(
        Wet[:f, 0].reshape(NW, f // NW)).reshape(-1)
    parts = sc_pet(xt.reshape(-1), src, dst, wet_pack)
    sv = (jnp.zeros((1, 128), jnp.float32)
          .at[0, 0].set(Wet[f, 0]).at[0, 1].set(Wet[f + 1, 0])
          .at[0, 2].set(bet[0]))
    pet = _tc_pet(parts.reshape(NW, e), edge_weight.T, sv).reshape(e, 1)

    p2t = _tc_layer_t(acc1.reshape(f, n), cnt, W1, b1, Wc2[:h], bc2)
    acc2 = sc_conv(p2t.reshape(-1), src, dst, ew_flat, _pack_wcb(Wc2, h, h // NW))

    h2, p3t = _tc_layer2_t(acc2.reshape(h, n), cnt, W2, b2, Wc3[:l], bc3,
                           latent=l)
    mu = h2[:, :l]
    logvar = h2[:, l:]
    acc3 = sc_conv_l(p3t.reshape(-1), src, dst, ew_flat, _pack_wcb(Wc3, l, l // NW))

    p4t = _tc_layer_t(acc3.reshape(l, n), cnt, W3, b3, Wc4[:h], bc4)
    acc4 = sc_conv(p4t.reshape(-1), src, dst, ew_flat, _pack_wcb(Wc4, h, h // NW))

    recon = _tc_final(acc4.reshape(h, n), cnt, W4, b4)
    return (recon, mu, logvar, pet)


# speedup vs baseline: 1.2251x; 1.2251x over previous
import jax
import jax.numpy as jnp
from jax import lax
from jax.experimental import pallas as pl
from jax.experimental.pallas import tpu as pltpu
from jax.experimental.pallas import tpu_sc as plsc

NC = 2
NS = 16
NW = NC * NS
LANES = 16
CHUNK = 4000
CCHUNK = 2000


def _sc_mesh():
    return plsc.VectorSubcoreMesh(
        core_axis_name="c", subcore_axis_name="s", num_cores=NC, num_subcores=NS
    )


def _full(s):
    return jnp.full((LANES,), s, jnp.float32)


def _make_sc_conv(n_nodes, n_edges, cpt, with_count):
    tw = n_nodes * cpt
    nch = n_edges // CHUNK
    assert nch % 2 == 0

    out_type = [jax.ShapeDtypeStruct((NW * tw,), jnp.float32)]
    scratch = [
        pltpu.VMEM((tw,), jnp.float32),
        pltpu.VMEM((tw,), jnp.float32),
        pltpu.VMEM((16,), jnp.float32),
    ]
    for _ in range(2):
        scratch += [pltpu.VMEM((CHUNK,), jnp.int32),
                    pltpu.VMEM((CHUNK,), jnp.int32),
                    pltpu.VMEM((2 * CHUNK,), jnp.float32)]
    scratch += [pltpu.SemaphoreType.DMA, pltpu.SemaphoreType.DMA]
    if with_count:
        out_type.append(jax.ShapeDtypeStruct((NW * n_nodes,), jnp.float32))
        scratch.append(pltpu.VMEM((n_nodes,), jnp.float32))

    def body(pt_hbm, src_hbm, dst_hbm, ew_hbm, wcb_hbm, *rest):
        if with_count:
            (acc_hbm, cnt_hbm, table_v, acc_v, wcb_v,
             sA, dA, eA, sB, dB, eB, semA, semB, cnt_v) = rest
        else:
            (acc_hbm, table_v, acc_v, wcb_v,
             sA, dA, eA, sB, dB, eB, semA, semB) = rest
        bufA = (sA, dA, eA)
        bufB = (sB, dB, eB)
        wid = lax.axis_index("s") * NC + lax.axis_index("c")
        pltpu.sync_copy(pt_hbm.at[pl.ds(wid * tw, tw)], table_v)
        pltpu.sync_copy(wcb_hbm.at[pl.ds(wid * 16, 16)], wcb_v)
        zero = jnp.zeros((LANES,), jnp.float32)

        @plsc.parallel_loop(0, tw, step=LANES, unroll=8)
        def _(i):
            acc_v[pl.ds(i, LANES)] = zero

        wrow = wcb_v[...]
        w0 = [_full(wrow[c]) for c in range(cpt)]
        w1 = [_full(wrow[8 + c]) for c in range(cpt)]
        lane2 = lax.iota(jnp.int32, LANES) * 2

        def fire(off, bufs, sem):
            pltpu.async_copy(src_hbm.at[pl.ds(off, CHUNK)], bufs[0], sem)
            pltpu.async_copy(dst_hbm.at[pl.ds(off, CHUNK)], bufs[1], sem)
            pltpu.async_copy(ew_hbm.at[pl.ds(2 * off, 2 * CHUNK)], bufs[2], sem)

        def drain(bufs, sem):
            pltpu.make_async_copy(src_hbm.at[pl.ds(0, CHUNK)], bufs[0], sem).wait()
            pltpu.make_async_copy(dst_hbm.at[pl.ds(0, CHUNK)], bufs[1], sem).wait()
            pltpu.make_async_copy(ew_hbm.at[pl.ds(0, 2 * CHUNK)], bufs[2], sem).wait()

        def process(bufs):
            @plsc.parallel_loop(0, CHUNK, step=LANES, unroll=4)
            def _(b):
                sv = bufs[0][pl.ds(b, LANES)]
                dv = bufs[1][pl.ds(b, LANES)]
                ei = lane2 + 2 * b
                e0 = plsc.load_gather(bufs[2], [ei])
                e1 = plsc.load_gather(bufs[2], [ei + 1])
                for c in range(cpt):
                    gth = plsc.load_gather(table_v, [sv + (c * n_nodes)])
                    m = jnp.maximum(gth + (e0 * w0[c] + e1 * w1[c]), 0.0)
                    plsc.addupdate_scatter(acc_v, [dv + (c * n_nodes)], m)

        last = (nch - 1) * CHUNK
        fire(0, bufA, semA)

        @pl.loop(0, nch // 2)
        def _(gp):
            g0 = gp * 2
            fire(jnp.minimum((g0 + 1) * CHUNK, last), bufB, semB)
            drain(bufA, semA)
            process(bufA)
            fire(jnp.minimum((g0 + 2) * CHUNK, last), bufA, semA)
            drain(bufB, semB)
            process(bufB)

        drain(bufA, semA)

        if with_count:
            @plsc.parallel_loop(0, n_nodes, step=LANES, unroll=8)
            def _(i):
                cnt_v[pl.ds(i, LANES)] = zero

            epw = n_edges // NW
            base = wid * epw
            ones = jnp.ones((LANES,), jnp.float32)

            @pl.loop(0, epw // CCHUNK)
            def _(g):
                db = dA.at[pl.ds(0, CCHUNK)]
                pltpu.sync_copy(dst_hbm.at[pl.ds(base + g * CCHUNK, CCHUNK)], db)

                @plsc.parallel_loop(0, CCHUNK, step=LANES, unroll=4)
                def _(i):
                    plsc.addupdate_scatter(cnt_v, [dA[pl.ds(i, LANES)]], ones)

            pltpu.sync_copy(cnt_v, cnt_hbm.at[pl.ds(wid * n_nodes, n_nodes)])

        pltpu.sync_copy(acc_v, acc_hbm.at[pl.ds(wid * tw, tw)])

    return pl.kernel(
        body,
        out_type=tuple(out_type) if with_count else out_type[0],
        mesh=_sc_mesh(),
        scratch_types=tuple(scratch),
        compiler_params=pltpu.CompilerParams(needs_layout_passes=False),
    )


def _make_sc_pet(n_nodes, n_edges):
    cpt = 4
    tw = n_nodes * cpt
    nch = n_edges // CHUNK
    assert nch % 2 == 0

    def body(xt_hbm, src_hbm, dst_hbm, wet_hbm, out_hbm,
             table_v, wet_v, sA, dA, sB, dB, oA, oB, semA, semB, semWA, semWB):
        bufA = (sA, dA)
        bufB = (sB, dB)
        srcs = (src_hbm, dst_hbm)
        wid = lax.axis_index("s") * NC + lax.axis_index("c")
        obase = wid * n_edges
        pltpu.sync_copy(xt_hbm.at[pl.ds(wid * tw, tw)], table_v)
        pltpu.sync_copy(wet_hbm.at[pl.ds(wid * 16, 16)], wet_v)
        wetrow = wet_v[...]
        wv = [_full(wetrow[c]) for c in range(cpt)]

        def fire(off, bufs, sem):
            for hb, b in zip(srcs, bufs):
                pltpu.async_copy(hb.at[pl.ds(off, CHUNK)], b, sem)

        def drain(bufs, sem):
            for hb, b in zip(srcs, bufs):
                pltpu.make_async_copy(hb.at[pl.ds(0, CHUNK)], b, sem).wait()

        def process(bufs, ob):
            @plsc.parallel_loop(0, CHUNK, step=LANES, unroll=4)
            def _(b):
                sv = bufs[0][pl.ds(b, LANES)]
                dv = bufs[1][pl.ds(b, LANES)]
                acc = jnp.zeros((LANES,), jnp.float32)
                for c in range(cpt):
                    a = plsc.load_gather(table_v, [sv + (c * n_nodes)])
                    bb = plsc.load_gather(table_v, [dv + (c * n_nodes)])
                    acc = acc + jnp.abs(a - bb) * wv[c]
                ob[pl.ds(b, LANES)] = acc

        def wdrain(ob, semw):
            pltpu.make_async_copy(src_hbm.at[pl.ds(0, CHUNK)], ob, semw).wait()

        last = (nch - 1) * CHUNK
        fire(0, bufA, semA)
        pltpu.async_copy(oA, out_hbm.at[pl.ds(obase, CHUNK)], semWA)
        pltpu.async_copy(oB, out_hbm.at[pl.ds(obase + CHUNK, CHUNK)], semWB)

        @pl.loop(0, nch // 2)
        def _(gp):
            g0 = gp * 2
            fire(jnp.minimum((g0 + 1) * CHUNK, last), bufB, semB)
            drain(bufA, semA)
            wdrain(oA, semWA)
            process(bufA, oA)
            pltpu.async_copy(oA, out_hbm.at[pl.ds(obase + g0 * CHUNK, CHUNK)], semWA)
            fire(jnp.minimum((g0 + 2) * CHUNK, last), bufA, semA)
            drain(bufB, semB)
            wdrain(oB, semWB)
            process(bufB, oB)
            pltpu.async_copy(oB, out_hbm.at[pl.ds(obase + (g0 + 1) * CHUNK, CHUNK)], semWB)

        drain(bufA, semA)
        wdrain(oA, semWA)
        wdrain(oB, semWB)

    return pl.kernel(
        body,
        out_type=jax.ShapeDtypeStruct((NW * n_edges,), jnp.float32),
        mesh=_sc_mesh(),
        scratch_types=(
            pltpu.VMEM((tw,), jnp.float32),
            pltpu.VMEM((16,), jnp.float32),
            pltpu.VMEM((CHUNK,), jnp.int32),
            pltpu.VMEM((CHUNK,), jnp.int32),
            pltpu.VMEM((CHUNK,), jnp.int32),
            pltpu.VMEM((CHUNK,), jnp.int32),
            pltpu.VMEM((CHUNK,), jnp.float32),
            pltpu.VMEM((CHUNK,), jnp.float32),
            pltpu.SemaphoreType.DMA, pltpu.SemaphoreType.DMA,
            pltpu.SemaphoreType.DMA, pltpu.SemaphoreType.DMA,
        ),
        compiler_params=pltpu.CompilerParams(needs_layout_passes=False),
    )


_TC_PARAMS = pltpu.CompilerParams(vmem_limit_bytes=100 * 1024 * 1024)


def _mm_tt(a, b):
    return lax.dot_general(a, b, (((0,), (0,)), ((), ())),
                           preferred_element_type=jnp.float32)


def _inv_cnt(cnt_ref):
    c = jnp.sum(cnt_ref[...], axis=0, keepdims=True)
    return 1.0 / jnp.maximum(c, 1.0)


def _tc_call(body, out_shapes, *args):
    outs = [jax.ShapeDtypeStruct(s, jnp.float32) for s in out_shapes]
    return pl.pallas_call(
        body,
        out_shape=outs[0] if len(outs) == 1 else outs,
        compiler_params=_TC_PARAMS,
    )(*args)


def _tc_project_t(x, wc, bc):
    n, f = x.shape
    dout = wc.shape[1]

    def body(x_ref, wc_ref, bc_ref, o_ref, xt_ref):
        xt = x_ref[...].T
        xt_ref[...] = xt
        o_ref[...] = _mm_tt(wc_ref[...], xt) + bc_ref[...]

    return _tc_call(body, [(dout, n), (f, n)], x, wc, bc.reshape(-1, 1))


def _tc_layer_t(acct, cnt, w, b, wc, bc):
    n = acct.shape[1]
    dout = wc.shape[1]

    def body(acc_ref, cnt_ref, w_ref, b_ref, wc_ref, bc_ref, o_ref):
        aggt = acc_ref[...] * _inv_cnt(cnt_ref)
        ht = jnp.maximum(_mm_tt(w_ref[...], aggt) + b_ref[...], 0.0)
        o_ref[...] = _mm_tt(wc_ref[...], ht) + bc_ref[...]

    return _tc_call(body, [(dout, n)], acct, cnt, w, b.reshape(-1, 1),
                    wc, bc.reshape(-1, 1))


def _tc_layer2_t(acct, cnt, w, b, wc, bc, latent):
    n = acct.shape[1]
    dmid = w.shape[1]
    dout = wc.shape[1]

    def body(acc_ref, cnt_ref, w_ref, b_ref, wc_ref, bc_ref, h_ref, p_ref):
        aggt = acc_ref[...] * _inv_cnt(cnt_ref)
        ht = jnp.maximum(_mm_tt(w_ref[...], aggt) + b_ref[...], 0.0)
        h_ref[...] = ht.T
        p_ref[...] = _mm_tt(wc_ref[...], ht[:latent]) + bc_ref[...]

    return _tc_call(body, [(n, dmid), (dout, n)], acct, cnt, w,
                    b.reshape(-1, 1), wc, bc.reshape(-1, 1))


def _tc_final(acct, cnt, w, b):
    n = acct.shape[1]
    dout = w.shape[1]

    def body(acc_ref, cnt_ref, w_ref, b_ref, o_ref):
        aggt = acc_ref[...] * _inv_cnt(cnt_ref)
        o_ref[...] = jnp.tanh(_mm_tt(aggt, w_ref[...]) + b_ref[...])

    return _tc_call(body, [(n, dout)], acct, cnt, w, b.reshape(1, -1))


def _tc_pet(parts, ewr, sv):
    e = parts.shape[1]
    be = 12800

    def body(p_ref, ew_ref, s_ref, o_ref):
        s = jnp.sum(p_ref[...], axis=0, keepdims=True)
        o_ref[...] = (s + ew_ref[0:1, :] * s_ref[0, 0]
                      + ew_ref[1:2, :] * s_ref[0, 1] + s_ref[0, 2])

    return pl.pallas_call(
        body,
        grid=(e // be,),
        in_specs=[pl.BlockSpec((NW, be), lambda i: (0, i)),
                  pl.BlockSpec((2, be), lambda i: (0, i)),
                  pl.BlockSpec((1, 128), lambda i: (0, 0))],
        out_specs=pl.BlockSpec((1, be), lambda i: (0, i)),
        out_shape=jax.ShapeDtypeStruct((1, e), jnp.float32),
        compiler_params=_TC_PARAMS,
    )(parts, ewr, sv)


def _pack_wcb(wc, din, cpt):
    bot = wc[din:]
    b0 = bot[0].reshape(NW, cpt)
    b1 = bot[1].reshape(NW, cpt)
    out = jnp.zeros((NW, 16), jnp.float32)
    return out.at[:, :cpt].set(b0).at[:, 8:8 + cpt].set(b1).reshape(-1)


def kernel(x, edge_index, edge_weight, W1, b1, Wc1, bc1, W2, b2, Wc2, bc2,
           W3, b3, Wc3, bc3, W4, b4, Wc4, bc4, Wet, bet):
    n, f = x.shape
    e = edge_index.shape[1]
    h = W1.shape[1]
    l = W3.shape[0]

    src = edge_index[0]
    dst = edge_index[1]
    ew_flat = edge_weight.reshape(-1)

    sc_conv_first = _make_sc_conv(n, e, f // NW, with_count=True)
    sc_conv = _make_sc_conv(n, e, f // NW, with_count=False)
    sc_conv_l = _make_sc_conv(n, e, l // NW, with_count=False)
    sc_pet = _make_sc_pet(n, e)

    p1t, xt = _tc_project_t(x, Wc1[:f], bc1)
    acc1, cntp = sc_conv_first(p1t.reshape(-1), src, dst, ew_flat,
                               _pack_wcb(Wc1, f, f // NW))
    cnt = cntp.reshape(NW, n)

    wet_pack = jnp.zeros((NW, 16), jnp.float32).at[:, :4].set(
        Wet[:f, 0].reshape(NW, f // NW)).reshape(-1)
    parts = sc_pet(xt.reshape(-1), src, dst, wet_pack)
    sv = (jnp.zeros((1, 128), jnp.float32)
          .at[0, 0].set(Wet[f, 0]).at[0, 1].set(Wet[f + 1, 0])
          .at[0, 2].set(bet[0]))
    pet = _tc_pet(parts.reshape(NW, e), edge_weight.T, sv).reshape(e, 1)

    p2t = _tc_layer_t(acc1.reshape(f, n), cnt, W1, b1, Wc2[:h], bc2)
    acc2 = sc_conv(p2t.reshape(-1), src, dst, ew_flat, _pack_wcb(Wc2, h, h // NW))

    h2, p3t = _tc_layer2_t(acc2.reshape(h, n), cnt, W2, b2, Wc3[:l], bc3,
                           latent=l)
    mu = h2[:, :l]
    logvar = h2[:, l:]
    acc3 = sc_conv_l(p3t.reshape(-1), src, dst, ew_flat, _pack_wcb(Wc3, l, l // NW))

    p4t = _tc_layer_t(acc3.reshape(l, n), cnt, W3, b3, Wc4[:h], bc4)
    acc4 = sc_conv(p4t.reshape(-1), src, dst, ew_flat, _pack_wcb(Wc4, h, h // NW))

    recon = _tc_final(acc4.reshape(h, n), cnt, W4, b4)
    return (recon, mu, logvar, pet)


# speedup vs baseline: 1.2265x; 1.0011x over previous
import jax
import jax.numpy as jnp
from jax import lax
from jax.experimental import pallas as pl
from jax.experimental.pallas import tpu as pltpu
from jax.experimental.pallas import tpu_sc as plsc

NC = 2
NS = 16
NW = NC * NS
LANES = 16
CHUNK = 4000
CCHUNK = 2000


def _sc_mesh():
    return plsc.VectorSubcoreMesh(
        core_axis_name="c", subcore_axis_name="s", num_cores=NC, num_subcores=NS
    )


def _full(s):
    return jnp.full((LANES,), s, jnp.float32)


def _make_sc_conv(n_nodes, n_edges, cpt, with_count):
    tw = n_nodes * cpt
    nch = n_edges // CHUNK
    assert nch % 2 == 0

    out_type = [jax.ShapeDtypeStruct((NW * tw,), jnp.float32)]
    scratch = [
        pltpu.VMEM((tw,), jnp.float32),
        pltpu.VMEM((tw,), jnp.float32),
        pltpu.VMEM((16,), jnp.float32),
    ]
    for _ in range(2):
        scratch += [pltpu.VMEM((CHUNK,), jnp.int32),
                    pltpu.VMEM((CHUNK,), jnp.int32),
                    pltpu.VMEM((2 * CHUNK,), jnp.float32)]
    scratch += [pltpu.SemaphoreType.DMA, pltpu.SemaphoreType.DMA]
    if with_count:
        out_type.append(jax.ShapeDtypeStruct((NW * n_nodes,), jnp.float32))
        scratch.append(pltpu.VMEM((n_nodes,), jnp.float32))

    def body(pt_hbm, src_hbm, dst_hbm, ew_hbm, wcb_hbm, *rest):
        if with_count:
            (acc_hbm, cnt_hbm, table_v, acc_v, wcb_v,
             sA, dA, eA, sB, dB, eB, semA, semB, cnt_v) = rest
        else:
            (acc_hbm, table_v, acc_v, wcb_v,
             sA, dA, eA, sB, dB, eB, semA, semB) = rest
        bufA = (sA, dA, eA)
        bufB = (sB, dB, eB)
        wid = lax.axis_index("s") * NC + lax.axis_index("c")
        pltpu.sync_copy(pt_hbm.at[pl.ds(wid * tw, tw)], table_v)
        pltpu.sync_copy(wcb_hbm.at[pl.ds(wid * 16, 16)], wcb_v)
        zero = jnp.zeros((LANES,), jnp.float32)

        @plsc.parallel_loop(0, tw, step=LANES, unroll=8)
        def _(i):
            acc_v[pl.ds(i, LANES)] = zero

        wrow = wcb_v[...]
        w0 = [_full(wrow[c]) for c in range(cpt)]
        w1 = [_full(wrow[8 + c]) for c in range(cpt)]
        lane2 = lax.iota(jnp.int32, LANES) * 2

        def fire(off, bufs, sem):
            pltpu.async_copy(src_hbm.at[pl.ds(off, CHUNK)], bufs[0], sem)
            pltpu.async_copy(dst_hbm.at[pl.ds(off, CHUNK)], bufs[1], sem)
            pltpu.async_copy(ew_hbm.at[pl.ds(2 * off, 2 * CHUNK)], bufs[2], sem)

        def drain(bufs, sem):
            pltpu.make_async_copy(src_hbm.at[pl.ds(0, CHUNK)], bufs[0], sem).wait()
            pltpu.make_async_copy(dst_hbm.at[pl.ds(0, CHUNK)], bufs[1], sem).wait()
            pltpu.make_async_copy(ew_hbm.at[pl.ds(0, 2 * CHUNK)], bufs[2], sem).wait()

        def process(bufs):
            @plsc.parallel_loop(0, CHUNK, step=LANES, unroll=4)
            def _(b):
                sv = bufs[0][pl.ds(b, LANES)]
                dv = bufs[1][pl.ds(b, LANES)]
                ei = lane2 + 2 * b
                e0 = plsc.load_gather(bufs[2], [ei])
                e1 = plsc.load_gather(bufs[2], [ei + 1])
                for c in range(cpt):
                    gth = plsc.load_gather(table_v, [sv + (c * n_nodes)])
                    m = jnp.maximum(gth + (e0 * w0[c] + e1 * w1[c]), 0.0)
                    plsc.addupdate_scatter(acc_v, [dv + (c * n_nodes)], m)

        last = (nch - 1) * CHUNK
        fire(0, bufA, semA)

        @pl.loop(0, nch // 2)
        def _(gp):
            g0 = gp * 2
            fire(jnp.minimum((g0 + 1) * CHUNK, last), bufB, semB)
            drain(bufA, semA)
            process(bufA)
            fire(jnp.minimum((g0 + 2) * CHUNK, last), bufA, semA)
            drain(bufB, semB)
            process(bufB)

        drain(bufA, semA)

        if with_count:
            @plsc.parallel_loop(0, n_nodes, step=LANES, unroll=8)
            def _(i):
                cnt_v[pl.ds(i, LANES)] = zero

            epw = n_edges // NW
            base = wid * epw
            ones = jnp.ones((LANES,), jnp.float32)

            @pl.loop(0, epw // CCHUNK)
            def _(g):
                db = dA.at[pl.ds(0, CCHUNK)]
                pltpu.sync_copy(dst_hbm.at[pl.ds(base + g * CCHUNK, CCHUNK)], db)

                @plsc.parallel_loop(0, CCHUNK, step=LANES, unroll=4)
                def _(i):
                    plsc.addupdate_scatter(cnt_v, [dA[pl.ds(i, LANES)]], ones)

            pltpu.sync_copy(cnt_v, cnt_hbm.at[pl.ds(wid * n_nodes, n_nodes)])

        pltpu.sync_copy(acc_v, acc_hbm.at[pl.ds(wid * tw, tw)])

    return pl.kernel(
        body,
        out_type=tuple(out_type) if with_count else out_type[0],
        mesh=_sc_mesh(),
        scratch_types=tuple(scratch),
        compiler_params=pltpu.CompilerParams(needs_layout_passes=False),
    )


def _make_sc_pet(n_nodes, n_edges):
    cpt = 4
    tw = n_nodes * cpt
    nch = n_edges // CHUNK
    assert nch % 2 == 0

    def body(xt_hbm, src_hbm, dst_hbm, wet_hbm, out_hbm,
             table_v, wet_v, sA, dA, sB, dB, oA, oB, semA, semB, semWA, semWB):
        bufA = (sA, dA)
        bufB = (sB, dB)
        srcs = (src_hbm, dst_hbm)
        wid = lax.axis_index("s") * NC + lax.axis_index("c")
        obase = wid * n_edges
        pltpu.sync_copy(xt_hbm.at[pl.ds(wid * tw, tw)], table_v)
        pltpu.sync_copy(wet_hbm.at[pl.ds(wid * 16, 16)], wet_v)
        wetrow = wet_v[...]
        wv = [_full(wetrow[c]) for c in range(cpt)]

        def fire(off, bufs, sem):
            for hb, b in zip(srcs, bufs):
                pltpu.async_copy(hb.at[pl.ds(off, CHUNK)], b, sem)

        def drain(bufs, sem):
            for hb, b in zip(srcs, bufs):
                pltpu.make_async_copy(hb.at[pl.ds(0, CHUNK)], b, sem).wait()

        def process(bufs, ob):
            @plsc.parallel_loop(0, CHUNK, step=LANES, unroll=4)
            def _(b):
                sv = bufs[0][pl.ds(b, LANES)]
                dv = bufs[1][pl.ds(b, LANES)]
                acc = jnp.zeros((LANES,), jnp.float32)
                for c in range(cpt):
                    a = plsc.load_gather(table_v, [sv + (c * n_nodes)])
                    bb = plsc.load_gather(table_v, [dv + (c * n_nodes)])
                    acc = acc + jnp.abs(a - bb) * wv[c]
                ob[pl.ds(b, LANES)] = acc

        def wdrain(ob, semw):
            pltpu.make_async_copy(src_hbm.at[pl.ds(0, CHUNK)], ob, semw).wait()

        last = (nch - 1) * CHUNK
        fire(0, bufA, semA)
        pltpu.async_copy(oA, out_hbm.at[pl.ds(obase, CHUNK)], semWA)
        pltpu.async_copy(oB, out_hbm.at[pl.ds(obase + CHUNK, CHUNK)], semWB)

        @pl.loop(0, nch // 2)
        def _(gp):
            g0 = gp * 2
            fire(jnp.minimum((g0 + 1) * CHUNK, last), bufB, semB)
            drain(bufA, semA)
            wdrain(oA, semWA)
            process(bufA, oA)
            pltpu.async_copy(oA, out_hbm.at[pl.ds(obase + g0 * CHUNK, CHUNK)], semWA)
            fire(jnp.minimum((g0 + 2) * CHUNK, last), bufA, semA)
            drain(bufB, semB)
            wdrain(oB, semWB)
            process(bufB, oB)
            pltpu.async_copy(oB, out_hbm.at[pl.ds(obase + (g0 + 1) * CHUNK, CHUNK)], semWB)

        drain(bufA, semA)
        wdrain(oA, semWA)
        wdrain(oB, semWB)

    return pl.kernel(
        body,
        out_type=jax.ShapeDtypeStruct((NW * n_edges,), jnp.float32),
        mesh=_sc_mesh(),
        scratch_types=(
            pltpu.VMEM((tw,), jnp.float32),
            pltpu.VMEM((16,), jnp.float32),
            pltpu.VMEM((CHUNK,), jnp.int32),
            pltpu.VMEM((CHUNK,), jnp.int32),
            pltpu.VMEM((CHUNK,), jnp.int32),
            pltpu.VMEM((CHUNK,), jnp.int32),
            pltpu.VMEM((CHUNK,), jnp.float32),
            pltpu.VMEM((CHUNK,), jnp.float32),
            pltpu.SemaphoreType.DMA, pltpu.SemaphoreType.DMA,
            pltpu.SemaphoreType.DMA, pltpu.SemaphoreType.DMA,
        ),
        compiler_params=pltpu.CompilerParams(needs_layout_passes=False),
    )


_TC_PARAMS = pltpu.CompilerParams(vmem_limit_bytes=100 * 1024 * 1024)


def _mm_tt(a, b):
    return lax.dot_general(a, b, (((0,), (0,)), ((), ())),
                           preferred_element_type=jnp.float32)


def _inv_cnt(cnt_ref):
    c = jnp.sum(cnt_ref[...], axis=0, keepdims=True)
    return 1.0 / jnp.maximum(c, 1.0)


def _tc_call(body, out_shapes, *args):
    outs = [jax.ShapeDtypeStruct(s, jnp.float32) for s in out_shapes]
    return pl.pallas_call(
        body,
        out_shape=outs[0] if len(outs) == 1 else outs,
        compiler_params=_TC_PARAMS,
    )(*args)


def _tc_project_t(x, wc, bc):
    n, f = x.shape
    dout = wc.shape[1]

    def body(x_ref, wc_ref, bc_ref, o_ref, xt_ref):
        x = x_ref[...]
        eye = (lax.broadcasted_iota(jnp.int32, (f, f), 0)
               == lax.broadcasted_iota(jnp.int32, (f, f), 1)).astype(jnp.float32)
        xt_ref[...] = lax.dot_general(eye, x, (((1,), (1,)), ((), ())),
                                      preferred_element_type=jnp.float32)
        o_ref[...] = lax.dot_general(wc_ref[...], x, (((0,), (1,)), ((), ())),
                                     preferred_element_type=jnp.float32) + bc_ref[...]

    return _tc_call(body, [(dout, n), (f, n)], x, wc, bc.reshape(-1, 1))


def _tc_layer_t(acct, cnt, w, b, wc, bc):
    n = acct.shape[1]
    dout = wc.shape[1]

    def body(acc_ref, cnt_ref, w_ref, b_ref, wc_ref, bc_ref, o_ref):
        aggt = acc_ref[...] * _inv_cnt(cnt_ref)
        ht = jnp.maximum(_mm_tt(w_ref[...], aggt) + b_ref[...], 0.0)
        o_ref[...] = _mm_tt(wc_ref[...], ht) + bc_ref[...]

    return _tc_call(body, [(dout, n)], acct, cnt, w, b.reshape(-1, 1),
                    wc, bc.reshape(-1, 1))


def _tc_layer2_t(acct, cnt, w, b, wc, bc, latent):
    n = acct.shape[1]
    dmid = w.shape[1]
    dout = wc.shape[1]

    def body(acc_ref, cnt_ref, w_ref, b_ref, wc_ref, bc_ref, h_ref, p_ref):
        aggt = acc_ref[...] * _inv_cnt(cnt_ref)
        ht = jnp.maximum(_mm_tt(w_ref[...], aggt) + b_ref[...], 0.0)
        h_ref[...] = ht.T
        p_ref[...] = _mm_tt(wc_ref[...], ht[:latent]) + bc_ref[...]

    return _tc_call(body, [(n, dmid), (dout, n)], acct, cnt, w,
                    b.reshape(-1, 1), wc, bc.reshape(-1, 1))


def _tc_final(acct, cnt, w, b):
    n = acct.shape[1]
    dout = w.shape[1]

    def body(acc_ref, cnt_ref, w_ref, b_ref, o_ref):
        aggt = acc_ref[...] * _inv_cnt(cnt_ref)
        o_ref[...] = jnp.tanh(_mm_tt(aggt, w_ref[...]) + b_ref[...])

    return _tc_call(body, [(n, dout)], acct, cnt, w, b.reshape(1, -1))


def _tc_pet(parts, ewr, sv):
    e = parts.shape[1]
    be = 12800

    def body(p_ref, ew_ref, s_ref, o_ref):
        s = jnp.sum(p_ref[...], axis=0, keepdims=True)
        o_ref[...] = (s + ew_ref[0:1, :] * s_ref[0, 0]
                      + ew_ref[1:2, :] * s_ref[0, 1] + s_ref[0, 2])

    return pl.pallas_call(
        body,
        grid=(e // be,),
        in_specs=[pl.BlockSpec((NW, be), lambda i: (0, i)),
                  pl.BlockSpec((2, be), lambda i: (0, i)),
                  pl.BlockSpec((1, 128), lambda i: (0, 0))],
        out_specs=pl.BlockSpec((1, be), lambda i: (0, i)),
        out_shape=jax.ShapeDtypeStruct((1, e), jnp.float32),
        compiler_params=_TC_PARAMS,
    )(parts, ewr, sv)


def _pack_wcb(wc, din, cpt):
    bot = wc[din:]
    b0 = bot[0].reshape(NW, cpt)
    b1 = bot[1].reshape(NW, cpt)
    out = jnp.zeros((NW, 16), jnp.float32)
    return out.at[:, :cpt].set(b0).at[:, 8:8 + cpt].set(b1).reshape(-1)


def kernel(x, edge_index, edge_weight, W1, b1, Wc1, bc1, W2, b2, Wc2, bc2,
           W3, b3, Wc3, bc3, W4, b4, Wc4, bc4, Wet, bet):
    n, f = x.shape
    e = edge_index.shape[1]
    h = W1.shape[1]
    l = W3.shape[0]

    src = edge_index[0]
    dst = edge_index[1]
    ew_flat = edge_weight.reshape(-1)

    sc_conv_first = _make_sc_conv(n, e, f // NW, with_count=True)
    sc_conv = _make_sc_conv(n, e, f // NW, with_count=False)
    sc_conv_l = _make_sc_conv(n, e, l // NW, with_count=False)
    sc_pet = _make_sc_pet(n, e)

    p1t, xt = _tc_project_t(x, Wc1[:f], bc1)
    acc1, cntp = sc_conv_first(p1t.reshape(-1), src, dst, ew_flat,
                               _pack_wcb(Wc1, f, f // NW))
    cnt = cntp.reshape(NW, n)

    wet_pack = jnp.zeros((NW, 16), jnp.float32).at[:, :4].set(
        Wet[:f, 0].reshape(NW, f // NW)).reshape(-1)
    parts = sc_pet(xt.reshape(-1), src, dst, wet_pack)
    sv = (jnp.zeros((1, 128), jnp.float32)
          .at[0, 0].set(Wet[f, 0]).at[0, 1].set(Wet[f + 1, 0])
          .at[0, 2].set(bet[0]))
    pet = _tc_pet(parts.reshape(NW, e), edge_weight.T, sv).reshape(e, 1)

    p2t = _tc_layer_t(acc1.reshape(f, n), cnt, W1, b1, Wc2[:h], bc2)
    acc2 = sc_conv(p2t.reshape(-1), src, dst, ew_flat, _pack_wcb(Wc2, h, h // NW))

    h2, p3t = _tc_layer2_t(acc2.reshape(h, n), cnt, W2, b2, Wc3[:l], bc3,
                           latent=l)
    mu = h2[:, :l]
    logvar = h2[:, l:]
    acc3 = sc_conv_l(p3t.reshape(-1), src, dst, ew_flat, _pack_wcb(Wc3, l, l // NW))

    p4t = _tc_layer_t(acc3.reshape(l, n), cnt, W3, b3, Wc4[:h], bc4)
    acc4 = sc_conv(p4t.reshape(-1), src, dst, ew_flat, _pack_wcb(Wc4, h, h // NW))

    recon = _tc_final(acc4.reshape(h, n), cnt, W4, b4)
    return (recon, mu, logvar, pet)
